# baseline (speedup 1.0000x reference)
import functools
import jax
import jax.numpy as jnp
from jax import lax
from jax.experimental import pallas as pl
from jax.experimental.pallas import tpu as pltpu
from jax.experimental.pallas import tpu_sc as plsc

D_MODEL = 64
SCALE = 8.0
NUM_CORES = 2
NUM_SUBCORES = 16
NUM_WORKERS = NUM_CORES * NUM_SUBCORES
CHUNK = 320
NBUF = 2


@functools.partial(jax.jit, static_argnames=("n_idx",))
def _emb_lookup(x_flat, lut, n_idx):
    per_worker = n_idx // NUM_WORKERS
    n_chunks = per_worker // CHUNK
    n_steps = n_chunks // NBUF
    mesh = plsc.VectorSubcoreMesh(core_axis_name="c", subcore_axis_name="s")

    @functools.partial(
        pl.kernel,
        mesh=mesh,
        out_type=jax.ShapeDtypeStruct((n_idx, D_MODEL), jnp.float32),
        scratch_types=[
            pltpu.VMEM((per_worker,), jnp.int32),
            pltpu.VMEM((NBUF, CHUNK, D_MODEL), jnp.float32),
            pltpu.VMEM((NBUF, CHUNK, D_MODEL), jnp.float32),
            pltpu.SemaphoreType.DMA,
            pltpu.SemaphoreType.DMA,
            pltpu.SemaphoreType.DMA,
            pltpu.SemaphoreType.DMA,
        ],
        compiler_params=pltpu.CompilerParams(use_tc_tiling_on_sc=False),
    )
    def body(x_hbm, lut_hbm, out_hbm, idx_v, gbuf, sbuf, g0, g1, s0, s1):
        gsems = (g0, g1)
        ssems = (s0, s1)
        wid = lax.axis_index("s") * NUM_CORES + lax.axis_index("c")
        base = wid * per_worker
        pltpu.sync_copy(x_hbm.at[pl.ds(base, per_worker)], idx_v)

        def gather(g, b):
            return pltpu.make_async_copy(
                lut_hbm.at[idx_v.at[pl.ds(g * CHUNK, CHUNK)]], gbuf.at[b], gsems[b]
            )

        def scatter(g, b):
            return pltpu.make_async_copy(
                sbuf.at[b], out_hbm.at[pl.ds(base + g * CHUNK, CHUNK)], ssems[b]
            )

        for b in range(NBUF):
            gather(b, b).start()

        def step(s2, carry):
            for b in range(NBUF):
                g = s2 * NBUF + b
                gather(g, b).wait()

                @pl.when(s2 > 0)
                def _wait_prev_scatter():
                    scatter(g - NBUF, b).wait()

                def scale_row(r, c):
                    for j in range(D_MODEL // 16):
                        sbuf[b, r, pl.ds(16 * j, 16)] = (
                            gbuf[b, r, pl.ds(16 * j, 16)] * SCALE
                        )
                    return c

                lax.fori_loop(0, CHUNK, scale_row, 0, unroll=4)
                scatter(g, b).start()

                @pl.when(s2 < n_steps - 1)
                def _start_next_gather():
                    gather(g + NBUF, b).start()

            return carry

        lax.fori_loop(0, n_steps, step, 0)
        for b in range(NBUF):
            scatter(n_chunks - NBUF + b, b).wait()

    return body(x_flat, lut)


def kernel(x, lut):
    n_idx = x.shape[0] * x.shape[1]
    x_flat = x.reshape(n_idx)
    out = _emb_lookup(x_flat, lut, n_idx)
    return out.reshape(x.shape[0], x.shape[1], D_MODEL)


# speedup vs baseline: 1.0972x; 1.0972x over previous
import functools
import jax
import jax.numpy as jnp
from jax import lax
from jax.experimental import pallas as pl
from jax.experimental.pallas import tpu as pltpu
from jax.experimental.pallas import tpu_sc as plsc

D_MODEL = 64
SCALE = 8.0
NUM_CORES = 2
NUM_SUBCORES = 16
NUM_WORKERS = NUM_CORES * NUM_SUBCORES
CHUNK = 128


@functools.partial(jax.jit, static_argnames=("n_idx",))
def _emb_lookup(x_flat, lut_pairs, n_idx):
    per_worker = n_idx // NUM_WORKERS
    n_chunks = per_worker // CHUNK
    mesh = plsc.VectorSubcoreMesh(core_axis_name="c", subcore_axis_name="s")

    @functools.partial(
        pl.kernel,
        mesh=mesh,
        out_type=jax.ShapeDtypeStruct((n_idx, D_MODEL), jnp.float32),
        scratch_types=[
            pltpu.VMEM((2, CHUNK), jnp.int32),
            pltpu.VMEM((2, CHUNK), jnp.int32),
            pltpu.VMEM((2, CHUNK), jnp.int32),
            pltpu.VMEM((2, CHUNK, 128), jnp.float32),
            pltpu.VMEM((2, CHUNK, D_MODEL), jnp.float32),
            pltpu.SemaphoreType.DMA,
            pltpu.SemaphoreType.DMA,
            pltpu.SemaphoreType.DMA,
            pltpu.SemaphoreType.DMA,
            pltpu.SemaphoreType.DMA,
            pltpu.SemaphoreType.DMA,
        ],
        compiler_params=pltpu.CompilerParams(use_tc_tiling_on_sc=True),
    )
    def body(x_hbm, lut_hbm, out_hbm, idxb, pairb, halfb, gbuf, sbuf,
             i0, i1, g0, g1, o0, o1):
        isems = (i0, i1)
        gsems = (g0, g1)
        osems = (o0, o1)
        wid = lax.axis_index("s") * NUM_CORES + lax.axis_index("c")
        base = wid * per_worker

        def idx_copy(g, b):
            return pltpu.make_async_copy(
                x_hbm.at[pl.ds(base + g * CHUNK, CHUNK)], idxb.at[b], isems[b]
            )

        def gather(b):
            return pltpu.make_async_copy(
                lut_hbm.at[pairb.at[b]], gbuf.at[b], gsems[b]
            )

        def out_copy(g, b):
            return pltpu.make_async_copy(
                sbuf.at[b], out_hbm.at[pl.ds(base + g * CHUNK, CHUNK)], osems[b]
            )

        idx_copy(0, 0).start()
        idx_copy(1, 1).start()

        def sel_scale(br):
            def sel_grp(t, c):
                hv = halfb[br, pl.ds(t * 16, 16)]
                for rr in range(16):
                    r = t * 16 + rr
                    h = hv[rr]
                    for m in range(D_MODEL // 16):
                        sbuf[br, r, pl.ds(16 * m, 16)] = (
                            gbuf[br, r, pl.ds(h + 16 * m, 16)] * SCALE
                        )
                return c

            lax.fori_loop(0, CHUNK // 16, sel_grp, 0)

        def step(s2, carry):
            for b in range(2):
                g = s2 * 2 + b
                idx_copy(g, b).wait()

                def split16(t, c):
                    v = idxb[b, pl.ds(t * 16, 16)]
                    pairb[b, pl.ds(t * 16, 16)] = lax.shift_right_logical(v, 1)
                    halfb[b, pl.ds(t * 16, 16)] = (
                        lax.bitwise_and(v, 1) * D_MODEL
                    )
                    return c

                lax.fori_loop(0, CHUNK // 16, split16, 0, unroll=4)
                gather(b).start()

                @pl.when(g + 2 < n_chunks)
                def _():
                    idx_copy(g + 2, b).start()

                bp = 1 - b

                @pl.when(g > 0)
                def _():
                    gather(bp).wait()

                    @pl.when(g > 2)
                    def _():
                        out_copy(g - 3, bp).wait()

                    sel_scale(bp)
                    out_copy(g - 1, bp).start()

            return carry

        lax.fori_loop(0, n_chunks // 2, step, 0)

        bl = (n_chunks - 1) % 2
        gather(bl).wait()
        out_copy(n_chunks - 3, bl).wait()
        sel_scale(bl)
        out_copy(n_chunks - 1, bl).start()
        out_copy(n_chunks - 2, 1 - bl).wait()
        out_copy(n_chunks - 1, bl).wait()

    return body(x_flat, lut_pairs)


def kernel(x, lut):
    n_idx = x.shape[0] * x.shape[1]
    x_flat = x.reshape(n_idx)
    lut_pairs = lut.reshape(lut.shape[0] // 2, 2 * lut.shape[1])
    out = _emb_lookup(x_flat, lut_pairs, n_idx)
    return out.reshape(x.shape[0], x.shape[1], D_MODEL)


# speedup vs baseline: 1.1884x; 1.0831x over previous
import functools
import jax
import jax.numpy as jnp
from jax import lax
from jax.experimental import pallas as pl
from jax.experimental.pallas import tpu as pltpu
from jax.experimental.pallas import tpu_sc as plsc

D_MODEL = 64
SCALE = 8.0
NUM_CORES = 2
NUM_SUBCORES = 16
NUM_WORKERS = NUM_CORES * NUM_SUBCORES
CHUNK = 128


@functools.partial(jax.jit, static_argnames=("n_idx",))
def _emb_lookup(x_flat, lut_padded, n_idx):
    per_worker = n_idx // NUM_WORKERS
    n_chunks = per_worker // CHUNK
    mesh = plsc.VectorSubcoreMesh(core_axis_name="c", subcore_axis_name="s")

    @functools.partial(
        pl.kernel,
        mesh=mesh,
        out_type=jax.ShapeDtypeStruct((n_idx, D_MODEL), jnp.float32),
        scratch_types=[
            pltpu.VMEM((2, CHUNK), jnp.int32),
            pltpu.VMEM((2, CHUNK, 128), jnp.float32),
            pltpu.VMEM((2, CHUNK, D_MODEL), jnp.float32),
            pltpu.SemaphoreType.DMA,
            pltpu.SemaphoreType.DMA,
            pltpu.SemaphoreType.DMA,
            pltpu.SemaphoreType.DMA,
            pltpu.SemaphoreType.DMA,
            pltpu.SemaphoreType.DMA,
        ],
        compiler_params=pltpu.CompilerParams(use_tc_tiling_on_sc=True),
    )
    def body(x_hbm, lut_hbm, out_hbm, idxb, gbuf, sbuf,
             i0, i1, g0, g1, o0, o1):
        isems = (i0, i1)
        gsems = (g0, g1)
        osems = (o0, o1)
        wid = lax.axis_index("s") * NUM_CORES + lax.axis_index("c")
        base = wid * per_worker

        def idx_copy(g, b):
            return pltpu.make_async_copy(
                x_hbm.at[pl.ds(base + g * CHUNK, CHUNK)], idxb.at[b], isems[b]
            )

        def gather(b):
            return pltpu.make_async_copy(
                lut_hbm.at[idxb.at[b]], gbuf.at[b], gsems[b]
            )

        def out_copy(g, b):
            return pltpu.make_async_copy(
                sbuf.at[b],
                out_hbm.at[pl.ds(base + g * CHUNK, CHUNK)],
                osems[b],
            )

        def scale(br):
            def scale_row(r, c):
                for m in range(D_MODEL // 16):
                    sbuf[br, r, pl.ds(16 * m, 16)] = (
                        gbuf[br, r, pl.ds(16 * m, 16)] * SCALE
                    )
                return c

            lax.fori_loop(0, CHUNK, scale_row, 0, unroll=4)

        idx_copy(0, 0).start()
        idx_copy(1, 1).start()

        def step(s2, carry):
            for b in range(2):
                g = s2 * 2 + b
                idx_copy(g, b).wait()

                @pl.when(g > 1)
                def _():
                    out_copy(g - 2, b).wait()

                gather(b).start()

                @pl.when(g + 2 < n_chunks)
                def _():
                    idx_copy(g + 2, b).start()

                bp = 1 - b

                @pl.when(g > 0)
                def _():
                    gather(bp).wait()
                    scale(bp)
                    out_copy(g - 1, bp).start()

            return carry

        lax.fori_loop(0, n_chunks // 2, step, 0)

        bl = (n_chunks - 1) % 2
        gather(bl).wait()
        scale(bl)
        out_copy(n_chunks - 1, bl).start()
        out_copy(n_chunks - 2, 1 - bl).wait()
        out_copy(n_chunks - 1, bl).wait()

    return body(x_flat, lut_padded)


def kernel(x, lut):
    n_idx = x.shape[0] * x.shape[1]
    x_flat = x.reshape(n_idx)
    lut_padded = jnp.pad(lut, ((0, 0), (0, 128 - D_MODEL)))
    out = _emb_lookup(x_flat, lut_padded, n_idx)
    return out.reshape(x.shape[0], x.shape[1], D_MODEL)
